# baseline (speedup 1.0000x reference)
import functools

import jax
import jax.numpy as jnp
from jax import lax
from jax.experimental import pallas as pl
from jax.experimental.pallas import tpu as pltpu
from jax.experimental.pallas import tpu_sc as plsc

N = 10000
E = 320000
D_IN = 128
D = 256

NC = 2
NS = 16
LANES = 16

GCOLS = 4
NGRP = D // GCOLS
GPW = NGRP // (NC * NS)
ECH = 4000
NECH = E // ECH
EPW = E // (NC * NS)


@functools.cache
def _mesh():
  return plsc.VectorSubcoreMesh(
      core_axis_name="c", subcore_axis_name="s", num_cores=NC,
      num_subcores=NS)


def _deg_body(dst_flat, out, dst_v, hist_v):
  c = lax.axis_index("c")
  s = lax.axis_index("s")
  wid = s * NC + c

  def zero_body(i, _):
    hist_v[pl.ds(i * LANES, LANES)] = jnp.zeros((LANES,), jnp.float32)
    return 0

  lax.fori_loop(0, N // LANES, zero_body, 0)
  pltpu.sync_copy(dst_flat.at[pl.ds(wid * EPW, EPW)], dst_v)

  ones = jnp.ones((LANES,), jnp.float32)

  def body(i, _):
    idx = dst_v[pl.ds(i * LANES, LANES)]
    plsc.addupdate_scatter(hist_v, [idx], ones)
    return 0

  lax.fori_loop(0, EPW // LANES, body, 0)
  pltpu.sync_copy(hist_v, out.at[wid, 0])


@functools.cache
def _deg_kernel():
  return pl.kernel(
      _deg_body,
      out_type=jax.ShapeDtypeStruct((NC * NS, 1, N), jnp.float32),
      mesh=_mesh(),
      scratch_types=[
          pltpu.VMEM((EPW,), jnp.int32),
          pltpu.VMEM((N,), jnp.float32),
      ],
      compiler_params=pltpu.CompilerParams(needs_layout_passes=False),
  )


def _agg_body(hpt, src_r, dst_r, out, t0, t1, t2, t3, a0, a1, a2, a3, src_v,
              dst_v, sem):
  c = lax.axis_index("c")
  s = lax.axis_index("s")
  widx = c * NS + s
  tabs = (t0, t1, t2, t3)
  accs = (a0, a1, a2, a3)

  for g in range(GPW):
    gb = g * (NC * NS) + widx
    for ci in range(GCOLS):
      pltpu.sync_copy(hpt.at[gb, ci, 0], tabs[ci])

    def zero_body(i, _):
      for ci in range(GCOLS):
        accs[ci][pl.ds(i * LANES, LANES)] = jnp.zeros((LANES,), jnp.float32)
      return 0

    lax.fori_loop(0, N // LANES, zero_body, 0)

    def idx_copy(ch, par):
      return (
          pltpu.make_async_copy(
              src_r.at[ch, 0], src_v.at[pl.ds(par * ECH, ECH)], sem),
          pltpu.make_async_copy(
              dst_r.at[ch, 0], dst_v.at[pl.ds(par * ECH, ECH)], sem),
      )

    for cp in idx_copy(0, 0):
      cp.start()

    def chunk_body(ch, _):
      par = lax.rem(ch, 2)
      for cp in idx_copy(ch, par):
        cp.wait()

      @pl.when(ch + 1 < NECH)
      def _():
        for cp in idx_copy(ch + 1, 1 - par):
          cp.start()

      base = par * ECH

      @plsc.parallel_loop(0, ECH // LANES, unroll=10)
      def _(v):
        o = base + v * LANES
        sv = src_v[pl.ds(o, LANES)]
        dv = dst_v[pl.ds(o, LANES)]
        for ci in range(GCOLS):
          vals = plsc.load_gather(tabs[ci], [sv])
          plsc.addupdate_scatter(accs[ci], [dv], vals)

      return 0

    lax.fori_loop(0, NECH, chunk_body, 0)
    for ci in range(GCOLS):
      pltpu.sync_copy(accs[ci], out.at[gb, ci, 0])


@functools.cache
def _agg_kernel():
  return pl.kernel(
      _agg_body,
      out_type=jax.ShapeDtypeStruct((NGRP, GCOLS, 1, N), jnp.float32),
      mesh=_mesh(),
      scratch_types=[pltpu.VMEM((N,), jnp.float32)] * (2 * GCOLS) + [
          pltpu.VMEM((2 * ECH,), jnp.int32),
          pltpu.VMEM((2 * ECH,), jnp.int32),
          pltpu.SemaphoreType.DMA,
      ],
      compiler_params=pltpu.CompilerParams(needs_layout_passes=False),
  )


def _dinv_body(parts_ref, o_ref):
  parts = jnp.squeeze(parts_ref[...], axis=1)
  ones = jnp.ones((1, NC * NS), jnp.float32)
  deg = lax.dot_general(ones, parts, (((1,), (0,)), ((), ())),
                        preferred_element_type=jnp.float32)
  o_ref[...] = lax.rsqrt(deg + 1.0)


def _tc_dinv(parts):
  return pl.pallas_call(
      _dinv_body,
      out_shape=jax.ShapeDtypeStruct((1, N), jnp.float32),
  )(parts)


def _mmt_body(xt_ref, w_ref, dinv_ref, o_ref):
  xs = xt_ref[...] * dinv_ref[...]
  ht = lax.dot_general(w_ref[...], xs, (((0,), (0,)), ((), ())),
                       preferred_element_type=jnp.float32)
  o_ref[...] = ht.reshape(o_ref.shape)


def _tc_mmt(xt, w, dinv_r):
  return pl.pallas_call(
      _mmt_body,
      out_shape=jax.ShapeDtypeStruct((NGRP, GCOLS, N), jnp.float32),
  )(xt, w, dinv_r)


def _comb_body(st_ref, hpt_ref, dinv_ref, b_ref, o_ref):
  tt = st_ref[...].reshape(D, N) + hpt_ref[...].reshape(D, N)
  o_ref[...] = tt * dinv_ref[...] + b_ref[...]


def _tc_comb(st, hpt, dinv_r, b_col):
  return pl.pallas_call(
      _comb_body,
      out_shape=jax.ShapeDtypeStruct((D, N), jnp.float32),
  )(st, hpt, dinv_r, b_col)


def _lnt_body(t_ref, g_ref, be_ref, fln_ref, fres_ref, xt_ref, o_ref):
  t = t_ref[...]
  ones = jnp.ones((1, D), jnp.float32)
  mu = lax.dot_general(ones, t, (((1,), (0,)), ((), ())),
                       preferred_element_type=jnp.float32) / D
  d = t - mu
  var = lax.dot_general(ones, d * d, (((1,), (0,)), ((), ())),
                        preferred_element_type=jnp.float32) / D
  y = d * lax.rsqrt(var + 1e-5) * g_ref[...] + be_ref[...]
  y = jnp.maximum(y, 0.0)
  fln = fln_ref[...]
  o_ref[...] = fln * y + (1.0 - fln) * t + fres_ref[...] * xt_ref[...]


def _tc_lnt(t, g_col, be_col, fln, fres, xt):
  return pl.pallas_call(
      _lnt_body,
      out_shape=jax.ShapeDtypeStruct((D, N), jnp.float32),
  )(t, g_col, be_col, fln, fres, xt)


def _trans_body(xt_ref, o_ref):
  o_ref[...] = xt_ref[...].T


def _tc_trans(xt):
  return pl.pallas_call(
      _trans_body,
      out_shape=jax.ShapeDtypeStruct((N, D), jnp.float32),
  )(xt)


def kernel(features, edge_index, W0, b0, W1, b1, W2, b2, g0, be0, g1, be1):
  src_r = edge_index[0].reshape(NECH, 1, ECH)
  dst_r = edge_index[1].reshape(NECH, 1, ECH)
  dst_flat = edge_index[1]

  parts = _deg_kernel()(dst_flat)
  dinv_r = _tc_dinv(parts)

  x0t = jnp.pad(features, ((0, 0), (0, D - D_IN))).T
  W0p = jnp.zeros((D, D), jnp.float32).at[:D_IN].set(W0)
  Ws = jnp.stack([W0p, W1, W2])
  bs = jnp.stack([b0, b1, b2]).reshape(3, D, 1)
  gs = jnp.stack([g0, g1, jnp.ones((D,), jnp.float32)]).reshape(3, D, 1)
  bes = jnp.stack([be0, be1, jnp.zeros((D,), jnp.float32)]).reshape(3, D, 1)
  flns = jnp.array([1.0, 1.0, 0.0], jnp.float32).reshape(3, 1, 1)
  fress = jnp.array([0.0, 1.0, 0.0], jnp.float32).reshape(3, 1, 1)

  def layer(xt, params):
    w, b, g, be, fln, fres = params
    hpt = _tc_mmt(xt, w, dinv_r)
    hpt4 = hpt.reshape(NGRP, GCOLS, 1, N)
    st4 = _agg_kernel()(hpt4, src_r, dst_r)
    st = st4.reshape(NGRP, GCOLS, N)
    t = _tc_comb(st, hpt, dinv_r, b)
    xt_next = _tc_lnt(t, g, be, fln, fres, xt)
    return xt_next, None

  xt_out, _ = lax.scan(layer, x0t, (Ws, bs, gs, bes, flns, fress))
  return _tc_trans(xt_out)


# speedup vs baseline: 1.0247x; 1.0247x over previous
import functools

import jax
import jax.numpy as jnp
from jax import lax
from jax.experimental import pallas as pl
from jax.experimental.pallas import tpu as pltpu
from jax.experimental.pallas import tpu_sc as plsc

N = 10000
E = 320000
D_IN = 128
D = 256

NC = 2
NS = 16
LANES = 16

GCOLS = 4
NGRP = D // GCOLS
GPW = NGRP // (NC * NS)
ECH = 4000
NECH = E // ECH
EPW = E // (NC * NS)


@functools.cache
def _mesh():
  return plsc.VectorSubcoreMesh(
      core_axis_name="c", subcore_axis_name="s", num_cores=NC,
      num_subcores=NS)


def _deg_body(dst_flat, out, dst_v, hist_v):
  c = lax.axis_index("c")
  s = lax.axis_index("s")
  wid = s * NC + c

  def zero_body(i, _):
    hist_v[pl.ds(i * LANES, LANES)] = jnp.zeros((LANES,), jnp.float32)
    return 0

  lax.fori_loop(0, N // LANES, zero_body, 0)
  pltpu.sync_copy(dst_flat.at[pl.ds(wid * EPW, EPW)], dst_v)

  ones = jnp.ones((LANES,), jnp.float32)

  def body(i, _):
    idx = dst_v[pl.ds(i * LANES, LANES)]
    plsc.addupdate_scatter(hist_v, [idx], ones)
    return 0

  lax.fori_loop(0, EPW // LANES, body, 0)
  pltpu.sync_copy(hist_v, out.at[wid, 0])


@functools.cache
def _deg_kernel():
  return pl.kernel(
      _deg_body,
      out_type=jax.ShapeDtypeStruct((NC * NS, 1, N), jnp.float32),
      mesh=_mesh(),
      scratch_types=[
          pltpu.VMEM((EPW,), jnp.int32),
          pltpu.VMEM((N,), jnp.float32),
      ],
      compiler_params=pltpu.CompilerParams(needs_layout_passes=False),
  )


def _agg_body(hpt, src_r, dst_r, out, t0, t1, t2, t3, a0, a1, a2, a3, src_v,
              dst_v, sem):
  c = lax.axis_index("c")
  s = lax.axis_index("s")
  widx = c * NS + s
  tabs = (t0, t1, t2, t3)
  accs = (a0, a1, a2, a3)

  for g in range(GPW):
    gb = g * (NC * NS) + widx
    for ci in range(GCOLS):
      pltpu.sync_copy(hpt.at[gb, ci, 0], tabs[ci])

    def zero_body(i, _):
      for ci in range(GCOLS):
        accs[ci][pl.ds(i * LANES, LANES)] = jnp.zeros((LANES,), jnp.float32)
      return 0

    lax.fori_loop(0, N // LANES, zero_body, 0)

    def idx_copy(ch, par):
      return (
          pltpu.make_async_copy(
              src_r.at[ch, 0], src_v.at[pl.ds(par * ECH, ECH)], sem),
          pltpu.make_async_copy(
              dst_r.at[ch, 0], dst_v.at[pl.ds(par * ECH, ECH)], sem),
      )

    for cp in idx_copy(0, 0):
      cp.start()

    def chunk_body(ch, _):
      par = lax.rem(ch, 2)
      for cp in idx_copy(ch, par):
        cp.wait()

      @pl.when(ch + 1 < NECH)
      def _():
        for cp in idx_copy(ch + 1, 1 - par):
          cp.start()

      base = par * ECH

      @plsc.parallel_loop(0, ECH // LANES, unroll=5)
      def _(v):
        o = base + v * LANES
        sv = src_v[pl.ds(o, LANES)]
        dv = dst_v[pl.ds(o, LANES)]
        for ci in range(GCOLS):
          vals = plsc.load_gather(tabs[ci], [sv])
          plsc.addupdate_scatter(accs[ci], [dv], vals)

      return 0

    lax.fori_loop(0, NECH, chunk_body, 0)
    for ci in range(GCOLS):
      pltpu.sync_copy(accs[ci], out.at[gb, ci, 0])


@functools.cache
def _agg_kernel():
  return pl.kernel(
      _agg_body,
      out_type=jax.ShapeDtypeStruct((NGRP, GCOLS, 1, N), jnp.float32),
      mesh=_mesh(),
      scratch_types=[pltpu.VMEM((N,), jnp.float32)] * (2 * GCOLS) + [
          pltpu.VMEM((2 * ECH,), jnp.int32),
          pltpu.VMEM((2 * ECH,), jnp.int32),
          pltpu.SemaphoreType.DMA,
      ],
      compiler_params=pltpu.CompilerParams(needs_layout_passes=False),
  )


def _dinv_body(parts_ref, o_ref):
  parts = jnp.squeeze(parts_ref[...], axis=1)
  ones = jnp.ones((1, NC * NS), jnp.float32)
  deg = lax.dot_general(ones, parts, (((1,), (0,)), ((), ())),
                        preferred_element_type=jnp.float32)
  o_ref[...] = lax.rsqrt(deg + 1.0)


def _tc_dinv(parts):
  return pl.pallas_call(
      _dinv_body,
      out_shape=jax.ShapeDtypeStruct((1, N), jnp.float32),
  )(parts)


def _mmt_body(xt_ref, w_ref, dinv_ref, o_ref):
  xs = xt_ref[...] * dinv_ref[...]
  ht = lax.dot_general(w_ref[...], xs, (((0,), (0,)), ((), ())),
                       preferred_element_type=jnp.float32)
  o_ref[...] = ht.reshape(o_ref.shape)


def _tc_mmt(xt, w, dinv_r):
  return pl.pallas_call(
      _mmt_body,
      out_shape=jax.ShapeDtypeStruct((NGRP, GCOLS, N), jnp.float32),
  )(xt, w, dinv_r)


def _comb_body(st_ref, hpt_ref, dinv_ref, b_ref, o_ref):
  tt = st_ref[...].reshape(D, N) + hpt_ref[...].reshape(D, N)
  o_ref[...] = tt * dinv_ref[...] + b_ref[...]


def _tc_comb(st, hpt, dinv_r, b_col):
  return pl.pallas_call(
      _comb_body,
      out_shape=jax.ShapeDtypeStruct((D, N), jnp.float32),
  )(st, hpt, dinv_r, b_col)


def _lnt_body(t_ref, g_ref, be_ref, fln_ref, fres_ref, xt_ref, o_ref):
  t = t_ref[...]
  ones = jnp.ones((1, D), jnp.float32)
  mu = lax.dot_general(ones, t, (((1,), (0,)), ((), ())),
                       preferred_element_type=jnp.float32) / D
  d = t - mu
  var = lax.dot_general(ones, d * d, (((1,), (0,)), ((), ())),
                        preferred_element_type=jnp.float32) / D
  y = d * lax.rsqrt(var + 1e-5) * g_ref[...] + be_ref[...]
  y = jnp.maximum(y, 0.0)
  fln = fln_ref[...]
  o_ref[...] = fln * y + (1.0 - fln) * t + fres_ref[...] * xt_ref[...]


def _tc_lnt(t, g_col, be_col, fln, fres, xt):
  return pl.pallas_call(
      _lnt_body,
      out_shape=jax.ShapeDtypeStruct((D, N), jnp.float32),
  )(t, g_col, be_col, fln, fres, xt)


def _trans_body(xt_ref, o_ref):
  o_ref[...] = xt_ref[...].T


def _tc_trans(xt):
  return pl.pallas_call(
      _trans_body,
      out_shape=jax.ShapeDtypeStruct((N, D), jnp.float32),
  )(xt)


def kernel(features, edge_index, W0, b0, W1, b1, W2, b2, g0, be0, g1, be1):
  src_r = edge_index[0].reshape(NECH, 1, ECH)
  dst_r = edge_index[1].reshape(NECH, 1, ECH)
  dst_flat = edge_index[1]

  parts = _deg_kernel()(dst_flat)
  dinv_r = _tc_dinv(parts)

  x0t = jnp.pad(features, ((0, 0), (0, D - D_IN))).T
  W0p = jnp.zeros((D, D), jnp.float32).at[:D_IN].set(W0)
  Ws = jnp.stack([W0p, W1, W2])
  bs = jnp.stack([b0, b1, b2]).reshape(3, D, 1)
  gs = jnp.stack([g0, g1, jnp.ones((D,), jnp.float32)]).reshape(3, D, 1)
  bes = jnp.stack([be0, be1, jnp.zeros((D,), jnp.float32)]).reshape(3, D, 1)
  flns = jnp.array([1.0, 1.0, 0.0], jnp.float32).reshape(3, 1, 1)
  fress = jnp.array([0.0, 1.0, 0.0], jnp.float32).reshape(3, 1, 1)

  def layer(xt, params):
    w, b, g, be, fln, fres = params
    hpt = _tc_mmt(xt, w, dinv_r)
    hpt4 = hpt.reshape(NGRP, GCOLS, 1, N)
    st4 = _agg_kernel()(hpt4, src_r, dst_r)
    st = st4.reshape(NGRP, GCOLS, N)
    t = _tc_comb(st, hpt, dinv_r, b)
    xt_next = _tc_lnt(t, g, be, fln, fres, xt)
    return xt_next, None

  xt_out, _ = lax.scan(layer, x0t, (Ws, bs, gs, bes, flns, fress))
  return _tc_trans(xt_out)
